# baseline (speedup 1.0000x reference)
import math

import jax
import jax.numpy as jnp
from jax.experimental import pallas as pl
from jax.experimental.pallas import tpu as pltpu

_D = 4096
_B = 8
_HID = 64
_TDIM = 128
_BN = 128
_NB = _D // _BN
_BN2 = 1024
_NB2 = _D // _BN2
_F32 = jnp.float32
_BF16 = jnp.bfloat16


def _coef(w0, w1, i, j):
    return jnp.sum(w0[:, i:i + 1] * w1[:, j:j + 1], axis=0, keepdims=True)


def _call1_body(ld_ref, lu_ref, xt_ref, t_ref, freqs_ref, tw1_ref, tb1_ref,
                tw2_ref, tb2_ref, w0_ref, w1_ref,
                lubf_ref, s1_ref, q_ref, tout_ref, coef_ref,
                ldbf_ref, u_ref, r_ref, p_ref):
    ph = pl.program_id(0)
    i = pl.program_id(1)
    base = i * _BN

    @pl.when((ph == 0) & (i == 0))
    def _prep():
        args = t_ref[...] * freqs_ref[...]
        cosr = jnp.cos(args)
        sinr = jnp.sin(args)
        h = jnp.dot(cosr, tw1_ref[0:_TDIM // 2, :], preferred_element_type=_F32)
        h = h + jnp.dot(sinr, tw1_ref[_TDIM // 2:_TDIM, :], preferred_element_type=_F32)
        h = h + tb1_ref[...]
        h = h * jax.lax.logistic(h)
        tout_ref[...] = jnp.dot(h, tw2_ref[...], preferred_element_type=_F32) + tb2_ref[...]
        w0 = w0_ref[...]
        w1 = w1_ref[...]
        c1 = _coef(w0, w1, 0, 0)
        c2 = _coef(w0, w1, 1, 0) + _coef(w0, w1, 0, 1)
        c3 = _coef(w0, w1, 2, 0) + _coef(w0, w1, 0, 2)
        c4 = _coef(w0, w1, 1, 1)
        c5 = _coef(w0, w1, 2, 1)
        c6 = _coef(w0, w1, 1, 2)
        c7 = _coef(w0, w1, 2, 2)
        coef_ref[...] = jnp.concatenate([c1, c2, c3, c4, c5, c6, c7, c1], axis=1)

    @pl.when(ph == 0)
    def _phase0():
        ld = ld_ref[...]
        u_ref[pl.ds(base, _BN), :] = jnp.dot(ld, xt_ref[...], preferred_element_type=_F32)
        ldbf_ref[pl.ds(base, _BN), :] = ld.astype(_BF16)

    @pl.when(ph == 1)
    def _phase1():
        lu = lu_ref[...]
        v = jnp.dot(lu, xt_ref[...], preferred_element_type=_F32)
        u = u_ref[pl.ds(base, _BN), :]
        c = coef_ref[...]
        r_ref[pl.ds(base, _BN), :] = c[0:1, 1:2] * u + c[0:1, 2:3] * v
        p_ref[pl.ds(base, _BN), :] = c[0:1, 3:4] * u + c[0:1, 4:5] * v
        q_ref[...] = c[0:1, 5:6] * u + c[0:1, 6:7] * v
        lubf_ref[...] = lu.astype(_BF16)

    @pl.when(ph == 2)
    def _phase2():
        c = coef_ref[...]
        sd = jnp.dot(ldbf_ref[pl.ds(base, _BN), :], p_ref[...].astype(_BF16),
                     preferred_element_type=_F32)
        s1_ref[...] = (c[0:1, 0:1] * xt_ref[pl.ds(base, _BN), :]
                       + r_ref[pl.ds(base, _BN), :]
                       + sd)


def _call2_body(lubf_ref, s1_ref, q_ref, mapwt_ref, mapb_ref, tout_ref,
                ow1_ref, ob1_ref, ow2_ref, ob2_ref, out_ref, acc_ref):
    i = pl.program_id(0)
    xo = s1_ref[...] + jnp.dot(lubf_ref[...], q_ref[...].astype(_BF16),
                               preferred_element_type=_F32)
    contrib = jnp.dot(mapwt_ref[...], xo, preferred_element_type=_F32)

    @pl.when(i == 0)
    def _init():
        acc_ref[...] = jnp.zeros_like(acc_ref)

    acc_ref[...] += contrib

    @pl.when(i == pl.num_programs(0) - 1)
    def _head():
        h = jnp.transpose(acc_ref[...]) + mapb_ref[...] + tout_ref[...]
        h = jnp.dot(h, ow1_ref[...], preferred_element_type=_F32) + ob1_ref[...]
        h = h * jax.lax.logistic(h)
        out_ref[...] = jnp.dot(h, ow2_ref[...], preferred_element_type=_F32) + ob2_ref[...]


def kernel(x, t, lap_down, lap_up, tW1, tb1, tW2, tb2, snn_w0, snn_w1,
           mapW, mapb, outW1, outb1, outW2, outb2):
    xt = x.T
    t2 = t.reshape(_B, 1)
    half = _TDIM // 2
    freqs = jnp.exp(
        -math.log(10000.0) * jnp.arange(0, half, dtype=_F32) / half
    ).reshape(1, half)
    w0r = snn_w0[0]
    w1r = snn_w1[:, 0, :]
    tb1r = tb1.reshape(1, _HID)
    tb2r = tb2.reshape(1, _HID)
    mapbr = mapb.reshape(1, _HID)
    ob1r = outb1.reshape(1, _HID)
    ob2r = outb2.reshape(1, _D)
    mapwt = mapW.T

    last = _NB - 1

    def ld_idx(ph, i):
        return (jnp.where(ph == 0, i, last), 0)

    def lu_idx(ph, i):
        return (jnp.where(ph == 0, 0, jnp.where(ph == 1, i, last)), 0)

    def lubf_idx(ph, i):
        return (jnp.where(ph == 0, 0, jnp.where(ph == 1, i, last)), 0)

    def s1_idx(ph, i):
        return (jnp.where(ph == 2, i, 0), 0)

    def q_idx(ph, i):
        return (jnp.where(ph == 0, 0, jnp.where(ph == 1, i, last)), 0)

    const2 = lambda ph, i: (0, 0)

    full = lambda shape: pl.BlockSpec(shape, const2)

    lubf, s1, q, tout, coefs = pl.pallas_call(
        _call1_body,
        grid=(3, _NB),
        in_specs=[
            pl.BlockSpec((_BN, _D), ld_idx),
            pl.BlockSpec((_BN, _D), lu_idx),
            full((_D, _B)),
            full((_B, 1)),
            full((1, half)),
            full((_TDIM, _HID)),
            full((1, _HID)),
            full((_HID, _HID)),
            full((1, _HID)),
            full((_HID, 3)),
            full((_HID, 3)),
        ],
        out_specs=(
            pl.BlockSpec((_BN, _D), lubf_idx),
            pl.BlockSpec((_BN, _B), s1_idx),
            pl.BlockSpec((_BN, _B), q_idx),
            full((_B, _HID)),
            full((1, 8)),
        ),
        out_shape=(
            jax.ShapeDtypeStruct((_D, _D), _BF16),
            jax.ShapeDtypeStruct((_D, _B), _F32),
            jax.ShapeDtypeStruct((_D, _B), _F32),
            jax.ShapeDtypeStruct((_B, _HID), _F32),
            jax.ShapeDtypeStruct((1, 8), _F32),
        ),
        scratch_shapes=[
            pltpu.VMEM((_D, _D), _BF16),
            pltpu.VMEM((_D, _B), _F32),
            pltpu.VMEM((_D, _B), _F32),
            pltpu.VMEM((_D, _B), _F32),
        ],
    )(lap_down, lap_up, xt, t2, freqs, tW1, tb1r, tW2, tb2r, w0r, w1r)

    blk2 = lambda i: (i, 0)
    out = pl.pallas_call(
        _call2_body,
        grid=(_NB2,),
        in_specs=[
            pl.BlockSpec((_BN2, _D), blk2),
            pl.BlockSpec((_BN2, _B), blk2),
            pl.BlockSpec((_D, _B), lambda i: (0, 0)),
            pl.BlockSpec((_HID, _BN2), lambda i: (0, i)),
            pl.BlockSpec((1, _HID), lambda i: (0, 0)),
            pl.BlockSpec((_B, _HID), lambda i: (0, 0)),
            pl.BlockSpec((_HID, _HID), lambda i: (0, 0)),
            pl.BlockSpec((1, _HID), lambda i: (0, 0)),
            pl.BlockSpec((_HID, _D), lambda i: (0, 0)),
            pl.BlockSpec((1, _D), lambda i: (0, 0)),
        ],
        out_specs=pl.BlockSpec((_B, _D), lambda i: (0, 0)),
        out_shape=jax.ShapeDtypeStruct((_B, _D), _F32),
        scratch_shapes=[pltpu.VMEM((_HID, _B), _F32)],
    )(lubf, s1, q, mapwt, mapbr, tout, outW1, ob1r, outW2, ob2r)
    return out


# speedup vs baseline: 1.2196x; 1.2196x over previous
import math

import jax
import jax.numpy as jnp
from jax.experimental import pallas as pl
from jax.experimental.pallas import tpu as pltpu

_D = 4096
_B = 8
_HID = 64
_TDIM = 128
_BN = 256
_NB = _D // _BN
_BN2 = 1024
_NB2 = _D // _BN2
_F32 = jnp.float32
_BF16 = jnp.bfloat16


def _coef(w0, w1, i, j):
    return jnp.sum(w0[:, i:i + 1] * w1[:, j:j + 1], axis=0, keepdims=True)


def _call1_body(ld_ref, lu_ref, xt_ref, t_ref, freqs_ref, tw1_ref, tb1_ref,
                tw2_ref, tb2_ref, w0_ref, w1_ref,
                s1_ref, q_ref, tout_ref, coef_ref,
                ldbf_ref, u_ref, r_ref, p_ref):
    ph = pl.program_id(0)
    i = pl.program_id(1)
    base = i * _BN

    @pl.when((ph == 0) & (i == 0))
    def _prep():
        args = t_ref[...] * freqs_ref[...]
        cosr = jnp.cos(args)
        sinr = jnp.sin(args)
        h = jnp.dot(cosr, tw1_ref[0:_TDIM // 2, :], preferred_element_type=_F32)
        h = h + jnp.dot(sinr, tw1_ref[_TDIM // 2:_TDIM, :], preferred_element_type=_F32)
        h = h + tb1_ref[...]
        h = h * jax.lax.logistic(h)
        tout_ref[...] = jnp.dot(h, tw2_ref[...], preferred_element_type=_F32) + tb2_ref[...]
        w0 = w0_ref[...]
        w1 = w1_ref[...]
        c1 = _coef(w0, w1, 0, 0)
        c2 = _coef(w0, w1, 1, 0) + _coef(w0, w1, 0, 1)
        c3 = _coef(w0, w1, 2, 0) + _coef(w0, w1, 0, 2)
        c4 = _coef(w0, w1, 1, 1)
        c5 = _coef(w0, w1, 2, 1)
        c6 = _coef(w0, w1, 1, 2)
        c7 = _coef(w0, w1, 2, 2)
        coef_ref[...] = jnp.concatenate([c1, c2, c3, c4, c5, c6, c7, c1], axis=1)

    @pl.when(ph == 0)
    def _phase0():
        ld = ld_ref[...]
        u_ref[pl.ds(base, _BN), :] = jnp.dot(ld, xt_ref[...], preferred_element_type=_F32)
        ldbf_ref[pl.ds(base, _BN), :] = ld.astype(_BF16)

    @pl.when(ph == 1)
    def _phase1():
        lu = lu_ref[...]
        v = jnp.dot(lu, xt_ref[...], preferred_element_type=_F32)
        u = u_ref[pl.ds(base, _BN), :]
        c = coef_ref[...]
        r_ref[pl.ds(base, _BN), :] = c[0:1, 1:2] * u + c[0:1, 2:3] * v
        p_ref[pl.ds(base, _BN), :] = c[0:1, 3:4] * u + c[0:1, 4:5] * v
        q_ref[...] = c[0:1, 5:6] * u + c[0:1, 6:7] * v

    @pl.when(ph == 2)
    def _phase2():
        c = coef_ref[...]
        sd = jnp.dot(ldbf_ref[pl.ds(base, _BN), :], p_ref[...].astype(_BF16),
                     preferred_element_type=_F32)
        s1_ref[...] = (c[0:1, 0:1] * xt_ref[pl.ds(base, _BN), :]
                       + r_ref[pl.ds(base, _BN), :]
                       + sd)


def _call2_body(lu_ref, s1_ref, q_ref, mapwt_ref, mapb_ref, tout_ref,
                ow1_ref, ob1_ref, ow2_ref, ob2_ref, out_ref, acc_ref):
    i = pl.program_id(0)
    xo = s1_ref[...] + jnp.dot(lu_ref[...], q_ref[...],
                               preferred_element_type=_F32)
    contrib = jnp.dot(mapwt_ref[...], xo, preferred_element_type=_F32)

    @pl.when(i == 0)
    def _init():
        acc_ref[...] = jnp.zeros_like(acc_ref)

    acc_ref[...] += contrib

    @pl.when(i == pl.num_programs(0) - 1)
    def _head():
        h = jnp.transpose(acc_ref[...]) + mapb_ref[...] + tout_ref[...]
        h = jnp.dot(h, ow1_ref[...], preferred_element_type=_F32) + ob1_ref[...]
        h = h * jax.lax.logistic(h)
        out_ref[...] = jnp.dot(h, ow2_ref[...], preferred_element_type=_F32) + ob2_ref[...]


def kernel(x, t, lap_down, lap_up, tW1, tb1, tW2, tb2, snn_w0, snn_w1,
           mapW, mapb, outW1, outb1, outW2, outb2):
    xt = x.T
    t2 = t.reshape(_B, 1)
    half = _TDIM // 2
    freqs = jnp.exp(
        -math.log(10000.0) * jnp.arange(0, half, dtype=_F32) / half
    ).reshape(1, half)
    w0r = snn_w0[0]
    w1r = snn_w1[:, 0, :]
    tb1r = tb1.reshape(1, _HID)
    tb2r = tb2.reshape(1, _HID)
    mapbr = mapb.reshape(1, _HID)
    ob1r = outb1.reshape(1, _HID)
    ob2r = outb2.reshape(1, _D)
    mapwt = mapW.T

    last = _NB - 1

    def ld_idx(ph, i):
        return (jnp.where(ph == 0, i, last), 0)

    def lu_idx(ph, i):
        return (jnp.where(ph == 0, 0, jnp.where(ph == 1, i, last)), 0)

    def s1_idx(ph, i):
        return (jnp.where(ph == 2, i, 0), 0)

    def q_idx(ph, i):
        return (jnp.where(ph == 0, 0, jnp.where(ph == 1, i, last)), 0)

    const2 = lambda ph, i: (0, 0)

    full = lambda shape: pl.BlockSpec(shape, const2)

    s1, q, tout, coefs = pl.pallas_call(
        _call1_body,
        grid=(3, _NB),
        in_specs=[
            pl.BlockSpec((_BN, _D), ld_idx),
            pl.BlockSpec((_BN, _D), lu_idx),
            full((_D, _B)),
            full((_B, 1)),
            full((1, half)),
            full((_TDIM, _HID)),
            full((1, _HID)),
            full((_HID, _HID)),
            full((1, _HID)),
            full((_HID, 3)),
            full((_HID, 3)),
        ],
        out_specs=(
            pl.BlockSpec((_BN, _B), s1_idx),
            pl.BlockSpec((_BN, _B), q_idx),
            full((_B, _HID)),
            full((1, 8)),
        ),
        out_shape=(
            jax.ShapeDtypeStruct((_D, _B), _F32),
            jax.ShapeDtypeStruct((_D, _B), _F32),
            jax.ShapeDtypeStruct((_B, _HID), _F32),
            jax.ShapeDtypeStruct((1, 8), _F32),
        ),
        scratch_shapes=[
            pltpu.VMEM((_D, _D), _BF16),
            pltpu.VMEM((_D, _B), _F32),
            pltpu.VMEM((_D, _B), _F32),
            pltpu.VMEM((_D, _B), _F32),
        ],
    )(lap_down, lap_up, xt, t2, freqs, tW1, tb1r, tW2, tb2r, w0r, w1r)

    blk2 = lambda i: (i, 0)
    out = pl.pallas_call(
        _call2_body,
        grid=(_NB2,),
        in_specs=[
            pl.BlockSpec((_BN2, _D), blk2),
            pl.BlockSpec((_BN2, _B), blk2),
            pl.BlockSpec((_D, _B), lambda i: (0, 0)),
            pl.BlockSpec((_HID, _BN2), lambda i: (0, i)),
            pl.BlockSpec((1, _HID), lambda i: (0, 0)),
            pl.BlockSpec((_B, _HID), lambda i: (0, 0)),
            pl.BlockSpec((_HID, _HID), lambda i: (0, 0)),
            pl.BlockSpec((1, _HID), lambda i: (0, 0)),
            pl.BlockSpec((_HID, _D), lambda i: (0, 0)),
            pl.BlockSpec((1, _D), lambda i: (0, 0)),
        ],
        out_specs=pl.BlockSpec((_B, _D), lambda i: (0, 0)),
        out_shape=jax.ShapeDtypeStruct((_B, _D), _F32),
        scratch_shapes=[pltpu.VMEM((_HID, _B), _F32)],
    )(lap_up, s1, q, mapwt, mapbr, tout, outW1, ob1r, outW2, ob2r)
    return out
